# baseline (speedup 1.0000x reference)
import functools

import jax
import jax.numpy as jnp
from jax.experimental import pallas as pl
from jax.experimental.pallas import tpu as pltpu
from jax.experimental.pallas import tpu_sc as plsc

_R1, _K1, _M1 = 0.2, 32, 512
_R2, _K2, _M2 = 0.4, 64, 128
_CB1 = 64
_CB2 = 64


def _cumsum_lanes(x):
    n = x.shape[-1]
    s = 1
    while s < n:
        pad = jnp.zeros(x.shape[:-1] + (s,), x.dtype)
        x = x + jnp.concatenate([pad, x[..., : n - s]], axis=-1)
        s *= 2
    return x


def _fps_body(M, xyz_ref, cen_ref, dist_ref):
    B, C, N = xyz_ref.shape
    xs = [xyz_ref[:, c, :] for c in range(C)]
    cen_ref[...] = jnp.zeros((B, C, M), jnp.float32)
    dist_ref[...] = jnp.full((B, N), 1e10, jnp.float32)
    iota_n = jax.lax.broadcasted_iota(jnp.int32, (B, N), 1)
    iota_m = jax.lax.broadcasted_iota(jnp.int32, (B, M), 1)

    def body(i, far):
        oh = (iota_n == far).astype(jnp.float32)
        ohm = (iota_m == i).astype(jnp.float32)
        s = None
        for c in range(C):
            cc = jnp.sum(xs[c] * oh, axis=1, keepdims=True)
            cen_ref[:, c, :] += cc * ohm
            d = xs[c] - cc
            s = d * d if s is None else s + d * d
        nd = jnp.minimum(dist_ref[...], s)
        dist_ref[...] = nd
        m = jnp.max(nd, axis=1, keepdims=True)
        far2 = jnp.min(jnp.where(nd == m, iota_n, N), axis=1, keepdims=True)
        return far2.astype(jnp.int32)

    jax.lax.fori_loop(0, M, body, jnp.zeros((B, 1), jnp.int32))


def _fps(xyzT, M):
    B, C, N = xyzT.shape
    return pl.pallas_call(
        functools.partial(_fps_body, M),
        out_shape=jax.ShapeDtypeStruct((B, C, M), jnp.float32),
        scratch_shapes=[pltpu.VMEM((B, N), jnp.float32)],
    )(xyzT)


def _bq_mlp_body(K, r2, cen_ref, xyzT_ref, feats_ref,
                 w1_ref, w1x_ref, b1_ref, w2_ref, b2_ref, w3_ref, b3_ref,
                 out_ref, fw_ref, sq_ref):
    CB, C = cen_ref.shape[1], cen_ref.shape[2]
    N = xyzT_ref.shape[2]
    xyzT = xyzT_ref[0]

    @pl.when(pl.program_id(1) == 0)
    def _():
        fw_ref[...] = jax.lax.dot_general(
            feats_ref[0], w1_ref[...], (((1,), (0,)), ((), ())),
            preferred_element_type=jnp.float32)
        sq_ref[...] = jnp.sum(xyzT * xyzT, axis=0, keepdims=True)

    cen = cen_ref[0]

    sq = sq_ref[...]
    censq = jnp.sum(cen * cen, axis=1, keepdims=True)
    prod = jax.lax.dot_general(cen, xyzT, (((1,), (0,)), ((), ())),
                               preferred_element_type=jnp.float32)
    dists = censq + sq - 2.0 * prod
    mask = dists <= jnp.float32(r2)
    mi = mask.astype(jnp.int32)
    csum = _cumsum_lanes(mi)
    rr = jnp.where(mask, csum, 0)
    count = csum[:, N - 1:N]

    kio = jax.lax.broadcasted_iota(jnp.int32, (CB, K, N), 1) + 1
    S = (rr[:, None, :] == kio).astype(jnp.float32)
    C1 = fw_ref.shape[1]
    t1 = jax.lax.dot_general(S.reshape(CB * K, N), fw_ref[...],
                             (((1,), (0,)), ((), ())),
                             preferred_element_type=jnp.float32)
    t2 = jax.lax.dot_general(cen, w1x_ref[...], (((1,), (0,)), ((), ())),
                             preferred_element_type=jnp.float32)
    h = t1.reshape(CB, K, C1) - t2[:, None, :] + b1_ref[...][:, None, :]
    h = jnp.maximum(h, 0.0).reshape(CB * K, C1)

    h = jax.lax.dot_general(h, w2_ref[...], (((1,), (0,)), ((), ())),
                            preferred_element_type=jnp.float32) + b2_ref[...]
    h = jnp.maximum(h, 0.0)
    h = jax.lax.dot_general(h, w3_ref[...], (((1,), (0,)), ((), ())),
                            preferred_element_type=jnp.float32) + b3_ref[...]
    h = jnp.maximum(h, 0.0)
    C3 = h.shape[1]
    kio2 = jax.lax.broadcasted_iota(jnp.int32, (CB, K, 1), 1)
    hm = jnp.where(kio2 < count[:, :, None], h.reshape(CB, K, C3), -1e30)
    out_ref[0] = jnp.max(hm, axis=1)


def _bq_mlp(cen, xyzT, feats, w1, w1x, b1, w2, b2, w3, b3, K, r2, CB):
    B, M, C = cen.shape
    N = xyzT.shape[2]
    F = feats.shape[2]
    C1, C2, C3 = w1.shape[1], w2.shape[1], w3.shape[1]
    b1r, b2r, b3r = b1.reshape(1, C1), b2.reshape(1, C2), b3.reshape(1, C3)
    grid = (B, M // CB)
    return pl.pallas_call(
        functools.partial(_bq_mlp_body, K, r2),
        grid=grid,
        in_specs=[
            pl.BlockSpec((1, CB, C), lambda b, j: (b, j, 0)),
            pl.BlockSpec((1, C, N), lambda b, j: (b, 0, 0)),
            pl.BlockSpec((1, N, F), lambda b, j: (b, 0, 0)),
            pl.BlockSpec((F, C1), lambda b, j: (0, 0)),
            pl.BlockSpec((C, C1), lambda b, j: (0, 0)),
            pl.BlockSpec((1, C1), lambda b, j: (0, 0)),
            pl.BlockSpec((C1, C2), lambda b, j: (0, 0)),
            pl.BlockSpec((1, C2), lambda b, j: (0, 0)),
            pl.BlockSpec((C2, C3), lambda b, j: (0, 0)),
            pl.BlockSpec((1, C3), lambda b, j: (0, 0)),
        ],
        out_specs=pl.BlockSpec((1, CB, C3), lambda b, j: (b, j, 0)),
        out_shape=jax.ShapeDtypeStruct((B, M, C3), jnp.float32),
        scratch_shapes=[pltpu.VMEM((N, C1), jnp.float32),
                        pltpu.VMEM((1, N), jnp.float32)],
    )(cen, xyzT, feats, w1, w1x, b1r, w2, b2r, w3, b3r)


def _bq_rank_body(K, r2, cen_ref, xyzT_ref, out_ref, sq_ref):
    CB, C = cen_ref.shape[1], cen_ref.shape[2]
    N = xyzT_ref.shape[2]
    xyzT = xyzT_ref[0]

    @pl.when(pl.program_id(1) == 0)
    def _():
        sq_ref[...] = jnp.sum(xyzT * xyzT, axis=0, keepdims=True)

    cen = cen_ref[0]
    censq = jnp.sum(cen * cen, axis=1, keepdims=True)
    prod = jax.lax.dot_general(cen, xyzT, (((1,), (0,)), ((), ())),
                               preferred_element_type=jnp.float32)
    dists = censq + sq_ref[...] - 2.0 * prod
    mask = dists <= jnp.float32(r2)
    csum = _cumsum_lanes(mask.astype(jnp.int32))
    rr = jnp.where(mask, csum, 0)

    kio = jax.lax.broadcasted_iota(jnp.int32, (CB, K, N), 1) + 1
    ion = jax.lax.broadcasted_iota(jnp.int32, (CB, K, N), 2)
    idxsel = jnp.min(jnp.where(rr[:, None, :] == kio, ion, N), axis=2)
    idx = jnp.where(idxsel < N, idxsel, idxsel[:, 0:1])
    out_ref[0] = idx + pl.program_id(0) * N


def _bq_rank(cen, xyzT, K, r2, CB):
    B, M, C = cen.shape
    N = xyzT.shape[2]
    return pl.pallas_call(
        functools.partial(_bq_rank_body, K, r2),
        grid=(B, M // CB),
        in_specs=[
            pl.BlockSpec((1, CB, C), lambda b, j: (b, j, 0)),
            pl.BlockSpec((1, C, N), lambda b, j: (b, 0, 0)),
        ],
        out_specs=pl.BlockSpec((1, CB, K), lambda b, j: (b, j, 0)),
        out_shape=jax.ShapeDtypeStruct((B, M, K), jnp.int32),
        scratch_shapes=[pltpu.VMEM((1, N), jnp.float32)],
    )(cen, xyzT)


def _fw_body(D, feats_ref, w1_ref, out_ref):
    t = jax.lax.dot_general(
        feats_ref[0], w1_ref[...], (((1,), (0,)), ((), ())),
        preferred_element_type=jnp.float32)
    N, C1 = t.shape
    out_ref[0] = jnp.concatenate(
        [t, jnp.zeros((N, D - C1), jnp.float32)], axis=1)


def _fw(feats, w1, D):
    B, N, F = feats.shape
    C1 = w1.shape[1]
    return pl.pallas_call(
        functools.partial(_fw_body, D),
        grid=(B,),
        in_specs=[
            pl.BlockSpec((1, N, F), lambda b: (b, 0, 0)),
            pl.BlockSpec((F, C1), lambda b: (0, 0)),
        ],
        out_specs=pl.BlockSpec((1, N, D), lambda b: (b, 0, 0)),
        out_shape=jax.ShapeDtypeStruct((B, N, D), jnp.float32),
    )(feats, w1)


def _sc_gather(table, idx):
    R = idx.shape[0]
    D = table.shape[1]
    info = plsc.get_sparse_core_info()
    NW = info.num_cores * info.num_subcores
    per_w = R // NW
    CH = 128
    n_ch = per_w // CH
    mesh = plsc.VectorSubcoreMesh(core_axis_name="c", subcore_axis_name="s")

    @functools.partial(
        pl.kernel, mesh=mesh,
        out_type=jax.ShapeDtypeStruct((R, D), jnp.float32),
        scratch_types=[
            pltpu.VMEM((per_w,), jnp.int32),
            pltpu.VMEM((CH, D), jnp.float32),
            pltpu.SemaphoreType.DMA,
        ],
    )
    def k(table_hbm, idx_hbm, out_hbm, idx_v, rows_v, sem):
        wid = jax.lax.axis_index("s") * info.num_cores + jax.lax.axis_index("c")
        base = wid * per_w
        pltpu.sync_copy(idx_hbm.at[pl.ds(base, per_w)], idx_v)

        def body(t, carry):
            off = pl.multiple_of(t * CH, 8)
            iv = idx_v.at[pl.ds(off, CH)]
            pltpu.async_copy(table_hbm.at[iv], rows_v, sem).wait()
            pltpu.sync_copy(rows_v, out_hbm.at[pl.ds(base + off, CH)])
            return carry

        jax.lax.fori_loop(0, n_ch, body, 0)

    return k(table, idx)


def _mlp_pool_body(K, C1, cen_ref, g_ref, w1x_ref, b1_ref, w2_ref, b2_ref,
                   w3_ref, b3_ref, out_ref):
    CB = cen_ref.shape[1]
    t1 = g_ref[0][:, 0:C1]
    t2 = jax.lax.dot_general(cen_ref[0], w1x_ref[...], (((1,), (0,)), ((), ())),
                             preferred_element_type=jnp.float32)
    h = t1.reshape(CB, K, C1) - t2[:, None, :] + b1_ref[...][:, None, :]
    h = jnp.maximum(h, 0.0).reshape(CB * K, C1)
    h = jax.lax.dot_general(h, w2_ref[...], (((1,), (0,)), ((), ())),
                            preferred_element_type=jnp.float32) + b2_ref[...]
    h = jnp.maximum(h, 0.0)
    h = jax.lax.dot_general(h, w3_ref[...], (((1,), (0,)), ((), ())),
                            preferred_element_type=jnp.float32) + b3_ref[...]
    h = jnp.maximum(h, 0.0)
    C3 = h.shape[1]
    out_ref[0] = jnp.max(h.reshape(CB, K, C3), axis=1)


def _mlp_pool(cen, g, w1x, b1, w2, b2, w3, b3, K, CB):
    B, M, C = cen.shape
    D = g.shape[2]
    C1 = w2.shape[0]
    C2, C3 = w2.shape[1], w3.shape[1]
    b1r, b2r, b3r = b1.reshape(1, C1), b2.reshape(1, C2), b3.reshape(1, C3)
    return pl.pallas_call(
        functools.partial(_mlp_pool_body, K, C1),
        grid=(B, M // CB),
        in_specs=[
            pl.BlockSpec((1, CB, C), lambda b, j: (b, j, 0)),
            pl.BlockSpec((1, CB * K, D), lambda b, j: (b, j, 0)),
            pl.BlockSpec((C, C1), lambda b, j: (0, 0)),
            pl.BlockSpec((1, C1), lambda b, j: (0, 0)),
            pl.BlockSpec((C1, C2), lambda b, j: (0, 0)),
            pl.BlockSpec((1, C2), lambda b, j: (0, 0)),
            pl.BlockSpec((C2, C3), lambda b, j: (0, 0)),
            pl.BlockSpec((1, C3), lambda b, j: (0, 0)),
        ],
        out_specs=pl.BlockSpec((1, CB, C3), lambda b, j: (b, j, 0)),
        out_shape=jax.ShapeDtypeStruct((B, M, C3), jnp.float32),
    )(cen, g, w1x, b1r, w2, b2r, w3, b3r)


def _mlp3_body(B, x_ref, w1_ref, b1_ref, w2_ref, b2_ref, w3_ref, b3_ref,
               out_ref):
    h = x_ref[...]
    h = jnp.maximum(jax.lax.dot_general(h, w1_ref[...], (((1,), (0,)), ((), ())),
                                        preferred_element_type=jnp.float32)
                    + b1_ref[...], 0.0)
    h = jnp.maximum(jax.lax.dot_general(h, w2_ref[...], (((1,), (0,)), ((), ())),
                                        preferred_element_type=jnp.float32)
                    + b2_ref[...], 0.0)
    h = jnp.maximum(jax.lax.dot_general(h, w3_ref[...], (((1,), (0,)), ((), ())),
                                        preferred_element_type=jnp.float32)
                    + b3_ref[...], 0.0)
    P, C3 = h.shape
    out_ref[...] = jnp.max(h.reshape(B, P // B, C3), axis=1)


def _mlp3(x, w1, b1, w2, b2, w3, b3, B):
    P, F = x.shape
    C1, C2, C3 = w1.shape[1], w2.shape[1], w3.shape[1]
    return pl.pallas_call(
        functools.partial(_mlp3_body, B),
        out_shape=jax.ShapeDtypeStruct((B, C3), jnp.float32),
    )(x, w1, b1.reshape(1, C1), w2, b2.reshape(1, C2), w3, b3.reshape(1, C3))


def kernel(points, w10, b10, w11, b11, w12, b12, w20, b20, w21, b21, w22, b22,
           w30, b30, w31, b31, w32, b32):
    B, C, N = points.shape
    xyzT = points

    cen1T = _fps(xyzT, _M1)
    cen1 = jnp.transpose(cen1T, (0, 2, 1))
    xyz_nat = jnp.transpose(points, (0, 2, 1))
    w1eff = w10[0:C, :] + w10[C:2 * C, :]
    D1 = 128
    fw1 = _fw(xyz_nat, w1eff, D1)
    gidx = _bq_rank(cen1, xyzT, K=_K1, r2=_R1 * _R1, CB=_CB1)
    g1 = _sc_gather(fw1.reshape(B * N, D1), gidx.reshape(B * _M1 * _K1))
    f1 = _mlp_pool(cen1, g1.reshape(B, _M1 * _K1, D1), w10[0:C, :], b10,
                   w11, b11, w12, b12, K=_K1, CB=_CB1)

    cen2T = _fps(cen1T, _M2)
    cen2 = jnp.transpose(cen2T, (0, 2, 1))
    feats2 = jnp.concatenate([cen1, f1], axis=-1)
    f2 = _bq_mlp(cen2, cen1T, feats2, w20, w20[0:C, :], b20,
                 w21, b21, w22, b22,
                 K=_K2, r2=_R2 * _R2, CB=_CB2)

    x3 = jnp.concatenate([cen2, f2], axis=-1).reshape(B * _M2, C + 256)
    f3 = _mlp3(x3, w30, b30, w31, b31, w32, b32, B)
    return f1, f2, f3.reshape(B, 1, 1024)


# speedup vs baseline: 1.4181x; 1.4181x over previous
import functools

import jax
import jax.numpy as jnp
from jax.experimental import pallas as pl
from jax.experimental.pallas import tpu as pltpu

_R1, _K1, _M1 = 0.2, 32, 512
_R2, _K2, _M2 = 0.4, 64, 128
_CB1 = 128
_CB2 = 128


def _cumsum_lanes(x):
    n = x.shape[-1]
    s = 1
    while s < n:
        pad = jnp.zeros(x.shape[:-1] + (s,), x.dtype)
        x = x + jnp.concatenate([pad, x[..., : n - s]], axis=-1)
        s *= 2
    return x


def _fps_body(M, xyz_ref, cen_ref, dist_ref):
    B, C, N = xyz_ref.shape
    xs = [xyz_ref[:, c, :] for c in range(C)]
    cen_ref[...] = jnp.zeros((B, C, M), jnp.float32)
    dist_ref[...] = jnp.full((B, N), 1e10, jnp.float32)
    iota_n = jax.lax.broadcasted_iota(jnp.int32, (B, N), 1)
    iota_m = jax.lax.broadcasted_iota(jnp.int32, (B, M), 1)

    def body(i, far):
        oh = (iota_n == far).astype(jnp.float32)
        ohm = (iota_m == i).astype(jnp.float32)
        s = None
        for c in range(C):
            cc = jnp.sum(xs[c] * oh, axis=1, keepdims=True)
            cen_ref[:, c, :] += cc * ohm
            d = xs[c] - cc
            s = d * d if s is None else s + d * d
        nd = jnp.minimum(dist_ref[...], s)
        dist_ref[...] = nd
        m = jnp.max(nd, axis=1, keepdims=True)
        far2 = jnp.min(jnp.where(nd == m, iota_n, N), axis=1, keepdims=True)
        return far2.astype(jnp.int32)

    jax.lax.fori_loop(0, M, body, jnp.zeros((B, 1), jnp.int32))


def _fps(xyzT, M):
    B, C, N = xyzT.shape
    return pl.pallas_call(
        functools.partial(_fps_body, M),
        out_shape=jax.ShapeDtypeStruct((B, C, M), jnp.float32),
        scratch_shapes=[pltpu.VMEM((B, N), jnp.float32)],
    )(xyzT)


def _bq_mlp_body(K, r2, cen_ref, xyzT_ref, feats_ref,
                 w1_ref, w1x_ref, b1_ref, w2_ref, b2_ref, w3_ref, b3_ref,
                 out_ref, fw_ref, sq_ref):
    CB, C = cen_ref.shape[1], cen_ref.shape[2]
    N = xyzT_ref.shape[2]
    xyzT = xyzT_ref[0]

    @pl.when(pl.program_id(1) == 0)
    def _():
        fw_ref[...] = jax.lax.dot_general(
            feats_ref[0], w1_ref[...], (((1,), (0,)), ((), ())),
            preferred_element_type=jnp.float32)
        sq_ref[...] = jnp.sum(xyzT * xyzT, axis=0, keepdims=True)

    cen = cen_ref[0]

    sq = sq_ref[...]
    censq = jnp.sum(cen * cen, axis=1, keepdims=True)
    prod = jax.lax.dot_general(cen, xyzT, (((1,), (0,)), ((), ())),
                               preferred_element_type=jnp.float32)
    dists = censq + sq - 2.0 * prod
    mask = dists <= jnp.float32(r2)
    mi = mask.astype(jnp.int32)
    csum = _cumsum_lanes(mi)
    rr = jnp.where(mask, csum, 0)
    count = csum[:, N - 1:N]

    kio = jax.lax.broadcasted_iota(jnp.int32, (CB, K, N), 1) + 1
    S = (rr[:, None, :] == kio).astype(jnp.float32)
    C1 = fw_ref.shape[1]
    t1 = jax.lax.dot_general(S.reshape(CB * K, N), fw_ref[...],
                             (((1,), (0,)), ((), ())),
                             preferred_element_type=jnp.float32)
    t2 = jax.lax.dot_general(cen, w1x_ref[...], (((1,), (0,)), ((), ())),
                             preferred_element_type=jnp.float32)
    h = t1.reshape(CB, K, C1) - t2[:, None, :] + b1_ref[...][:, None, :]
    h = jnp.maximum(h, 0.0).reshape(CB * K, C1)

    h = jax.lax.dot_general(h, w2_ref[...], (((1,), (0,)), ((), ())),
                            preferred_element_type=jnp.float32) + b2_ref[...]
    h = jnp.maximum(h, 0.0)
    h = jax.lax.dot_general(h, w3_ref[...], (((1,), (0,)), ((), ())),
                            preferred_element_type=jnp.float32) + b3_ref[...]
    h = jnp.maximum(h, 0.0)
    C3 = h.shape[1]
    kio2 = jax.lax.broadcasted_iota(jnp.int32, (CB, K, 1), 1)
    hm = jnp.where(kio2 < count[:, :, None], h.reshape(CB, K, C3), -1e30)
    out_ref[0] = jnp.max(hm, axis=1)


def _bq_mlp(cen, xyzT, feats, w1, w1x, b1, w2, b2, w3, b3, K, r2, CB):
    B, M, C = cen.shape
    N = xyzT.shape[2]
    F = feats.shape[2]
    C1, C2, C3 = w1.shape[1], w2.shape[1], w3.shape[1]
    b1r, b2r, b3r = b1.reshape(1, C1), b2.reshape(1, C2), b3.reshape(1, C3)
    grid = (B, M // CB)
    return pl.pallas_call(
        functools.partial(_bq_mlp_body, K, r2),
        grid=grid,
        in_specs=[
            pl.BlockSpec((1, CB, C), lambda b, j: (b, j, 0)),
            pl.BlockSpec((1, C, N), lambda b, j: (b, 0, 0)),
            pl.BlockSpec((1, N, F), lambda b, j: (b, 0, 0)),
            pl.BlockSpec((F, C1), lambda b, j: (0, 0)),
            pl.BlockSpec((C, C1), lambda b, j: (0, 0)),
            pl.BlockSpec((1, C1), lambda b, j: (0, 0)),
            pl.BlockSpec((C1, C2), lambda b, j: (0, 0)),
            pl.BlockSpec((1, C2), lambda b, j: (0, 0)),
            pl.BlockSpec((C2, C3), lambda b, j: (0, 0)),
            pl.BlockSpec((1, C3), lambda b, j: (0, 0)),
        ],
        out_specs=pl.BlockSpec((1, CB, C3), lambda b, j: (b, j, 0)),
        out_shape=jax.ShapeDtypeStruct((B, M, C3), jnp.float32),
        scratch_shapes=[pltpu.VMEM((N, C1), jnp.float32),
                        pltpu.VMEM((1, N), jnp.float32)],
    )(cen, xyzT, feats, w1, w1x, b1r, w2, b2r, w3, b3r)


def _mlp3_body(B, x_ref, w1_ref, b1_ref, w2_ref, b2_ref, w3_ref, b3_ref,
               out_ref):
    h = x_ref[...]
    h = jnp.maximum(jax.lax.dot_general(h, w1_ref[...], (((1,), (0,)), ((), ())),
                                        preferred_element_type=jnp.float32)
                    + b1_ref[...], 0.0)
    h = jnp.maximum(jax.lax.dot_general(h, w2_ref[...], (((1,), (0,)), ((), ())),
                                        preferred_element_type=jnp.float32)
                    + b2_ref[...], 0.0)
    h = jnp.maximum(jax.lax.dot_general(h, w3_ref[...], (((1,), (0,)), ((), ())),
                                        preferred_element_type=jnp.float32)
                    + b3_ref[...], 0.0)
    P, C3 = h.shape
    out_ref[...] = jnp.max(h.reshape(B, P // B, C3), axis=1)


def _mlp3(x, w1, b1, w2, b2, w3, b3, B):
    P, F = x.shape
    C1, C2, C3 = w1.shape[1], w2.shape[1], w3.shape[1]
    return pl.pallas_call(
        functools.partial(_mlp3_body, B),
        out_shape=jax.ShapeDtypeStruct((B, C3), jnp.float32),
    )(x, w1, b1.reshape(1, C1), w2, b2.reshape(1, C2), w3, b3.reshape(1, C3))


def kernel(points, w10, b10, w11, b11, w12, b12, w20, b20, w21, b21, w22, b22,
           w30, b30, w31, b31, w32, b32):
    B, C, N = points.shape
    xyzT = points

    cen1T = _fps(xyzT, _M1)
    cen1 = jnp.transpose(cen1T, (0, 2, 1))
    xyz_nat = jnp.transpose(points, (0, 2, 1))
    w1eff = w10[0:C, :] + w10[C:2 * C, :]
    f1 = _bq_mlp(cen1, xyzT, xyz_nat, w1eff, w10[0:C, :], b10,
                 w11, b11, w12, b12,
                 K=_K1, r2=_R1 * _R1, CB=_CB1)

    cen2T = _fps(cen1T, _M2)
    cen2 = jnp.transpose(cen2T, (0, 2, 1))
    feats2 = jnp.concatenate([cen1, f1], axis=-1)
    f2 = _bq_mlp(cen2, cen1T, feats2, w20, w20[0:C, :], b20,
                 w21, b21, w22, b22,
                 K=_K2, r2=_R2 * _R2, CB=_CB2)

    x3 = jnp.concatenate([cen2, f2], axis=-1).reshape(B * _M2, C + 256)
    f3 = _mlp3(x3, w30, b30, w31, b31, w32, b32, B)
    return f1, f2, f3.reshape(B, 1, 1024)
